# baseline (speedup 1.0000x reference)
import functools

import jax
import jax.numpy as jnp
from jax import lax
from jax.experimental import pallas as pl
from jax.experimental.pallas import tpu as pltpu
from jax.experimental.pallas import tpu_sc as plsc

N_NODES = 40962
N_OUT = 10242
K = 7
N_ROWS = 1024
NUM_WORKERS = 32
ROWS_PER_W = N_ROWS // NUM_WORKERS
ROW_TILES = (((N_NODES + 127) // 128 + 7) // 8) * 8
ROW_PAD = ROW_TILES * 128
OUT_TILES = (((N_OUT + 127) // 128 + 7) // 8) * 8
OUT_STRIDE = OUT_TILES * 128
VAL_TILES = (N_OUT + 127) // 128
QP = (VAL_TILES + 1) // 2
PK_ROWS = ((QP * K + 7) // 8) * 8
CHUNKS = ((0, 8, 16), (8, 16, 16), (16, 24, 16), (24, 32, 16),
          (32, 40, 16), (40, 41, 8))


@functools.partial(
    pl.kernel,
    mesh=plsc.VectorSubcoreMesh(core_axis_name="c", subcore_axis_name="s"),
    compiler_params=pltpu.CompilerParams(needs_layout_passes=False),
    out_type=jax.ShapeDtypeStruct((N_ROWS * OUT_TILES, 128), jnp.float32),
    scratch_types=[
        pltpu.VMEM((PK_ROWS, 128), jnp.float32),
        pltpu.VMEM((ROW_TILES, 128), jnp.float32),
        pltpu.VMEM((ROW_TILES, 128), jnp.float32),
        pltpu.VMEM((16, 128), jnp.float32),
        pltpu.VMEM((16, 128), jnp.float32),
        pltpu.SemaphoreType.DMA,
        pltpu.SemaphoreType.DMA,
        pltpu.SemaphoreType.DMA,
        pltpu.SemaphoreType.DMA,
    ],
)
def _pool(x_hbm, idx_hbm, out_hbm, idx_v, row_a, row_b, ob0, ob1,
          sem_a, sem_b, sem_o0, sem_o1):
    wid = lax.axis_index("s") * 2 + lax.axis_index("c")
    pltpu.sync_copy(idx_hbm, idx_v)
    scale = jnp.float32(1.0 / K)
    m16 = jnp.uint32(0xFFFF)
    m7 = jnp.uint32(127)
    obufs = (ob0, ob1)
    osems = (sem_o0, sem_o1)

    def process(row_v, row):
        pending = [None] * len(CHUNKS)
        for c, (qp0, qp1, nt) in enumerate(CHUNKS):
            buf = obufs[c % 2]
            if c >= 2:
                pending[c - 2].wait()

            def tile_qp(qp, c2, _qp0=qp0, _buf=buf):
                brow = 2 * (qp - _qp0)
                for e in range(2):
                    acc = [None] * 8
                    for k in range(K):
                        for h in range(4):
                            off = 448 * e + 64 * k + 16 * h
                            vecf = idx_v[7 * qp + off // 128,
                                         pl.ds(off % 128, 16)]
                            w = plsc.bitcast(vecf, jnp.uint32)
                            a = w & m16
                            b = w >> 16
                            va = plsc.load_gather(
                                row_v,
                                [plsc.bitcast(a >> 7, jnp.int32),
                                 plsc.bitcast(a & m7, jnp.int32)],
                            )
                            vb = plsc.load_gather(
                                row_v,
                                [plsc.bitcast(b >> 7, jnp.int32),
                                 plsc.bitcast(b & m7, jnp.int32)],
                            )
                            ia, ib = 2 * h, 2 * h + 1
                            if k == 0:
                                acc[ia], acc[ib] = va, vb
                            else:
                                acc[ia] = acc[ia] + va
                                acc[ib] = acc[ib] + vb
                    for h in range(4):
                        _buf[brow + e, pl.ds(32 * h, 16)] = (
                            acc[2 * h] * scale)
                        _buf[brow + e, pl.ds(32 * h + 16, 16)] = (
                            acc[2 * h + 1] * scale)
                return c2

            lax.fori_loop(qp0, qp1, tile_qp, 0)
            pending[c] = pltpu.async_copy(
                buf.at[pl.ds(0, nt)],
                out_hbm.at[pl.ds(row * OUT_TILES + 2 * qp0, nt)],
                osems[c % 2],
            )
        pending[len(CHUNKS) - 2].wait()
        pending[len(CHUNKS) - 1].wait()

    def pair_step(i, carry):
        r0 = wid * ROWS_PER_W + 2 * i
        h0 = pltpu.async_copy(
            x_hbm.at[pl.ds(r0 * ROW_TILES, ROW_TILES)], row_a, sem_a)
        h1 = pltpu.async_copy(
            x_hbm.at[pl.ds((r0 + 1) * ROW_TILES, ROW_TILES)], row_b, sem_b)
        h0.wait()
        process(row_a, r0)
        h1.wait()
        process(row_b, r0 + 1)
        return carry

    lax.fori_loop(0, ROWS_PER_W // 2, pair_step, 0)


def _pack_indices(neigh_orders):
    idx = neigh_orders[: N_OUT * K].astype(jnp.int32).reshape(N_OUT, K).T
    idx = jnp.pad(idx, ((0, 0), (0, QP * 256 - N_OUT)))
    a = idx.reshape(K, QP, 2, 4, 2, 16)
    packed = a[..., 0, :] | (a[..., 1, :] << 16)
    packed = packed.transpose(1, 2, 0, 3, 4).reshape(QP * K, 128)
    packed = jnp.pad(packed, ((0, PK_ROWS - QP * K), (0, 0)))
    return lax.bitcast_convert_type(packed, jnp.float32)


def kernel(x, neigh_orders):
    B, D, N = x.shape
    idx = _pack_indices(neigh_orders)
    x3 = x.reshape(B * D, N)
    head = x3[:, : 320 * 128].reshape(N_ROWS, 320, 128)
    tail = jnp.pad(x3[:, 320 * 128:], ((0, 0), (0, 126)))
    xp = jnp.concatenate(
        [head, tail.reshape(N_ROWS, 1, 128),
         jnp.zeros((N_ROWS, ROW_TILES - 321, 128), jnp.float32)], axis=1)
    xp = xp.reshape(N_ROWS * ROW_TILES, 128)
    out = _pool(xp, idx)
    out = out.reshape(N_ROWS, OUT_STRIDE)[:, :N_OUT]
    return out.reshape(B, D, N_OUT)


# speedup vs baseline: 1.2427x; 1.2427x over previous
import functools

import jax
import jax.numpy as jnp
from jax import lax
from jax.experimental import pallas as pl
from jax.experimental.pallas import tpu as pltpu
from jax.experimental.pallas import tpu_sc as plsc

N_NODES = 40962
N_OUT = 10242
K = 7
N_ROWS = 1024
NUM_WORKERS = 32
ROWS_PER_W = N_ROWS // NUM_WORKERS
ROW_TILES = (((N_NODES + 127) // 128 + 7) // 8) * 8
ROW_PAD = ROW_TILES * 128
OUT_TILES = (((N_OUT + 127) // 128 + 7) // 8) * 8
OUT_STRIDE = OUT_TILES * 128
VAL_TILES = (N_OUT + 127) // 128
QP = (VAL_TILES + 1) // 2
PK_ROWS = ((QP * K + 7) // 8) * 8
CHUNKS = ((0, 8, 16), (8, 16, 16), (16, 24, 16), (24, 32, 16),
          (32, 40, 16), (40, 41, 8))


@functools.partial(
    pl.kernel,
    mesh=plsc.VectorSubcoreMesh(core_axis_name="c", subcore_axis_name="s"),
    compiler_params=pltpu.CompilerParams(needs_layout_passes=False),
    out_type=jax.ShapeDtypeStruct((N_ROWS * OUT_TILES, 128), jnp.float32),
    scratch_types=[
        pltpu.VMEM((PK_ROWS, 128), jnp.float32),
        pltpu.VMEM((ROW_TILES, 128), jnp.float32),
        pltpu.VMEM((ROW_TILES, 128), jnp.float32),
        pltpu.VMEM((16, 128), jnp.float32),
        pltpu.VMEM((16, 128), jnp.float32),
        pltpu.SemaphoreType.DMA,
        pltpu.SemaphoreType.DMA,
        pltpu.SemaphoreType.DMA,
        pltpu.SemaphoreType.DMA,
    ],
)
def _pool(x_hbm, idx_hbm, out_hbm, idx_v, row_a, row_b, ob0, ob1,
          sem_a, sem_b, sem_o0, sem_o1):
    wid = lax.axis_index("s") * 2 + lax.axis_index("c")
    pltpu.sync_copy(idx_hbm, idx_v)
    scale = jnp.float32(1.0 / K)
    m16 = jnp.uint32(0xFFFF)
    m7 = jnp.uint32(127)
    obufs = (ob0, ob1)
    osems = (sem_o0, sem_o1)

    def process(row_v, row):
        pending = [None] * len(CHUNKS)
        for c, (qp0, qp1, nt) in enumerate(CHUNKS):
            buf = obufs[c % 2]
            if c >= 2:
                pending[c - 2].wait()

            def tile_qp(qp, c2, _qp0=qp0, _buf=buf):
                brow = 2 * (qp - _qp0)
                for e in range(2):
                    acc = [None] * 8
                    for k in range(K):
                        for h in range(4):
                            off = 448 * e + 64 * k + 16 * h
                            vecf = idx_v[7 * qp + off // 128,
                                         pl.ds(off % 128, 16)]
                            w = plsc.bitcast(vecf, jnp.uint32)
                            a = w & m16
                            b = w >> 16
                            va = plsc.load_gather(
                                row_v,
                                [plsc.bitcast(a >> 7, jnp.int32),
                                 plsc.bitcast(a & m7, jnp.int32)],
                            )
                            vb = plsc.load_gather(
                                row_v,
                                [plsc.bitcast(b >> 7, jnp.int32),
                                 plsc.bitcast(b & m7, jnp.int32)],
                            )
                            ia, ib = 2 * h, 2 * h + 1
                            if k == 0:
                                acc[ia], acc[ib] = va, vb
                            else:
                                acc[ia] = acc[ia] + va
                                acc[ib] = acc[ib] + vb
                    for h in range(4):
                        _buf[brow + e, pl.ds(32 * h, 16)] = (
                            acc[2 * h] * scale)
                        _buf[brow + e, pl.ds(32 * h + 16, 16)] = (
                            acc[2 * h + 1] * scale)
                return c2

            lax.fori_loop(qp0, qp1, tile_qp, 0)
            pending[c] = pltpu.async_copy(
                buf.at[pl.ds(0, nt)],
                out_hbm.at[pl.ds(row * OUT_TILES + 2 * qp0, nt)],
                osems[c % 2],
            )
        pending[len(CHUNKS) - 2].wait()
        pending[len(CHUNKS) - 1].wait()

    def pair_step(i, carry):
        r0 = wid * ROWS_PER_W + 2 * i
        h0 = pltpu.async_copy(
            x_hbm.at[pl.ds(r0 * ROW_TILES, ROW_TILES)], row_a, sem_a)
        h1 = pltpu.async_copy(
            x_hbm.at[pl.ds((r0 + 1) * ROW_TILES, ROW_TILES)], row_b, sem_b)
        h0.wait()
        process(row_a, r0)
        h1.wait()
        process(row_b, r0 + 1)
        return carry

    lax.fori_loop(0, ROWS_PER_W // 2, pair_step, 0)


def _pack_indices(neigh_orders):
    idx = neigh_orders[: N_OUT * K].astype(jnp.int32).reshape(N_OUT, K).T
    idx = jnp.pad(idx, ((0, 0), (0, QP * 256 - N_OUT)))
    a = idx.reshape(K, QP, 2, 4, 2, 16)
    packed = a[..., 0, :] | (a[..., 1, :] << 16)
    packed = packed.transpose(1, 2, 0, 3, 4).reshape(QP * K, 128)
    packed = jnp.pad(packed, ((0, PK_ROWS - QP * K), (0, 0)))
    return lax.bitcast_convert_type(packed, jnp.float32)


def kernel(x, neigh_orders):
    B, D, N = x.shape
    idx = _pack_indices(neigh_orders)
    xp = jnp.pad(x.reshape(B * D, N), ((0, 0), (0, ROW_PAD - N)))
    xp = xp.reshape(N_ROWS * ROW_TILES, 128)
    out = _pool(xp, idx)
    out = out.reshape(N_ROWS, OUT_STRIDE)[:, :N_OUT]
    return out.reshape(B, D, N_OUT)
